# baseline (speedup 1.0000x reference)
import math

import jax
import jax.numpy as jnp
from jax.experimental import pallas as pl
from jax.experimental.pallas import tpu as pltpu

NUM_TO_SAMPLE = 1024
K = 10
_ROWS = 256

_ACOS_C = (
    1.5707963050,
    -0.2145988016,
    0.0889789874,
    -0.0501743046,
    0.0308918810,
    -0.0170881256,
    0.0066700901,
    -0.0012624911,
)


def _fps_body(px_ref, py_ref, pz_ref, fmask_ref):
    H, N2 = px_ref.shape
    px = px_ref[...]
    py = py_ref[...]
    pz = pz_ref[...]
    lane = jax.lax.broadcasted_iota(jnp.int32, (H, N2), 1)
    sub = jax.lax.broadcasted_iota(jnp.int32, (H, N2), 0)
    iota_g = (lane + (sub % 2) * N2).astype(jnp.float32)
    evn = (jax.lax.broadcasted_iota(jnp.int32, (H, 1), 0) % 2) == 0
    nf = jnp.float32(2 * N2)

    def pair(v, op):
        up = pltpu.roll(v, 1, 0)
        dn = pltpu.roll(v, H - 1, 0)
        return op(v, jnp.where(evn, dn, up))

    def step(s, state):
        dists, far, fmask = state
        mask = iota_g == far
        fmask = jnp.where(mask, 1.0, fmask)
        cx = pair(jnp.sum(jnp.where(mask, px, 0.0), axis=1, keepdims=True),
                  jnp.add)
        cy = pair(jnp.sum(jnp.where(mask, py, 0.0), axis=1, keepdims=True),
                  jnp.add)
        cz = pair(jnp.sum(jnp.where(mask, pz, 0.0), axis=1, keepdims=True),
                  jnp.add)
        dx = px - cx
        dy = py - cy
        dz = pz - cz
        d = dx * dx + dy * dy + dz * dz
        dists = jnp.minimum(dists, d)
        m = pair(jnp.max(dists, axis=1, keepdims=True), jnp.maximum)
        far_new = pair(
            jnp.min(jnp.where(dists == m, iota_g, nf), axis=1, keepdims=True),
            jnp.minimum)
        return dists, far_new, fmask

    dists0 = jnp.full((H, N2), 1e10, dtype=jnp.float32)
    far0 = jnp.zeros((H, 1), dtype=jnp.float32)
    fmask0 = jnp.zeros((H, N2), dtype=jnp.float32)
    _, _, fmask = jax.lax.fori_loop(
        0, NUM_TO_SAMPLE, step, (dists0, far0, fmask0)
    )
    fmask_ref[...] = fmask


def _fps_pallas(pos):
    B, N, _ = pos.shape
    fmask = pl.pallas_call(
        _fps_body,
        out_shape=jax.ShapeDtypeStruct((B * 2, N // 2), jnp.float32),
    )(pos[:, :, 0].reshape(B * 2, N // 2),
      pos[:, :, 1].reshape(B * 2, N // 2),
      pos[:, :, 2].reshape(B * 2, N // 2))
    return fmask.reshape(B, N)


def _feat_body(px_ref, py_ref, pz_ref, bx_ref, by_ref, bz_ref, xx_ref,
               pxi_ref, pyi_ref, pzi_ref, bxi_ref, byi_ref, bzi_ref,
               xxi_ref, xf0_ref, xf1_ref, xf2_ref,
               ip_ref, d2_ref):
    R = _ROWS
    N = px_ref.shape[2]
    pxj = px_ref[0, :, :]
    pyj = py_ref[0, :, :]
    pzj = pz_ref[0, :, :]
    bxj = bx_ref[0, :, :]
    byj = by_ref[0, :, :]
    bzj = bz_ref[0, :, :]
    xxj = xx_ref[0, :, :]
    pxi = pxi_ref[0, :, :]
    pyi = pyi_ref[0, :, :]
    pzi = pzi_ref[0, :, :]
    bxi = bxi_ref[0, :, :]
    byi = byi_ref[0, :, :]
    bzi = bzi_ref[0, :, :]
    xxi = xxi_ref[0, :, :]
    xf0 = xf0_ref[0, :, :]
    xf1 = xf1_ref[0, :, :]
    xf2 = xf2_ref[0, :, :]

    dx = pxi - pxj
    dy = pyi - pyj
    dz = pzi - pzj
    d2 = dx * dx + dy * dy + dz * dz
    g = xf0 * pxj + xf1 * pyj + xf2 * pzj
    m3 = (bxi * bxj + byi * byj) + bzi * bzj
    inner = -2.0 * m3
    key = ((-xxi) - inner) - xxj

    iota = jax.lax.broadcasted_iota(jnp.int32, (R, N), 1).astype(jnp.float32)
    big = jnp.float32(3.4e38)
    nf = jnp.float32(N)
    keyw = key
    for t in range(K):
        m = jnp.max(keyw, axis=1, keepdims=True)
        amin = jnp.min(
            jnp.where(keyw == m, iota, nf), axis=1, keepdims=True
        )
        sel = iota == amin
        ip_ref[0, 0, :, t] = jnp.sum(jnp.where(sel, g, 0.0), axis=1)
        d2_ref[0, 0, :, t] = jnp.sum(jnp.where(sel, d2, 0.0), axis=1)
        keyw = jnp.where(sel, -big, keyw)


def _knn_feats(pos, x):
    B, N, _ = pos.shape
    nb = N // _ROWS
    grid = (B, nb)
    posb = pos.astype(jnp.bfloat16).astype(jnp.float32)
    xx = jnp.sum(jnp.swapaxes(pos, 1, 2) ** 2, axis=1)
    row_spec = pl.BlockSpec((1, 1, N), lambda b, r: (b, 0, 0))
    col_spec = pl.BlockSpec((1, _ROWS, 1), lambda b, r: (b, r, 0))
    out_spec = pl.BlockSpec((1, 1, _ROWS, K), lambda b, r: (b, r, 0, 0))
    oshape = jax.ShapeDtypeStruct((B, nb, _ROWS, K), jnp.float32)
    ip, d2 = pl.pallas_call(
        _feat_body,
        grid=grid,
        in_specs=[row_spec] * 7 + [col_spec] * 10,
        out_specs=(out_spec, out_spec),
        out_shape=(oshape, oshape),
    )(pos[:, None, :, 0], pos[:, None, :, 1], pos[:, None, :, 2],
      posb[:, None, :, 0], posb[:, None, :, 1], posb[:, None, :, 2],
      xx[:, None, :],
      pos[:, :, 0:1], pos[:, :, 1:2], pos[:, :, 2:3],
      posb[:, :, 0:1], posb[:, :, 1:2], posb[:, :, 2:3],
      xx[:, :, None],
      x[:, 0, :, None], x[:, 1, :, None], x[:, 2, :, None])
    return ip.reshape(B, N, K), d2.reshape(B, N, K)


_RRB = 512


def _rank_body(sm_row_ref, sm_col_ref, dl_col_ref, fm_col_ref,
               topi_ref, ps_ref):
    Rb = sm_col_ref.shape[1]
    N = sm_row_ref.shape[2]
    S = NUM_TO_SAMPLE
    rb = pl.program_id(1)
    sm_row = sm_row_ref[0, :, :]
    sm_col = sm_col_ref[0, :, :]
    dl_col = dl_col_ref[0, :, :]
    fm_col = fm_col_ref[0, :, :]

    iota_j = jax.lax.broadcasted_iota(jnp.int32, (1, N), 1).astype(jnp.float32)
    gidx = (jax.lax.broadcasted_iota(jnp.int32, (Rb, 1), 0)
            + rb * Rb).astype(jnp.float32)

    gt = jnp.where(sm_row > sm_col, 1.0, 0.0)
    tie = jnp.where((sm_row == sm_col) & (iota_j < gidx), 1.0, 0.0)
    rank = jnp.sum(gt + tie, axis=1, keepdims=True)

    iota_s = jax.lax.broadcasted_iota(jnp.int32, (1, S), 1).astype(jnp.float32)
    eqm = jnp.where(rank == iota_s, gidx, 0.0)
    contrib = jnp.sum(eqm, axis=0)

    @pl.when(rb == 0)
    def _():
        topi_ref[0, 0, :] = jnp.zeros((S,), dtype=jnp.float32)

    topi_ref[0, 0, :] = topi_ref[0, 0, :] + contrib

    selm = jnp.where(rank < S, 1.0, 0.0)
    shorl_p = jnp.sum(dl_col * selm)
    bdist_p = jnp.sum(dl_col * fm_col)
    total_p = jnp.sum(dl_col * sm_col)
    ps_ref[0, 0, 0, :] = jnp.stack([shorl_p, bdist_p, total_p])


def _rank_pallas(smax, dist_loss, fmask):
    B, N = smax.shape
    nb = N // _RRB
    grid = (B, nb)
    S = NUM_TO_SAMPLE
    row_spec = pl.BlockSpec((1, 1, N), lambda b, r: (b, 0, 0))
    col_spec = pl.BlockSpec((1, _RRB, 1), lambda b, r: (b, r, 0))
    topi_spec = pl.BlockSpec((1, 1, S), lambda b, r: (b, 0, 0))
    ps_spec = pl.BlockSpec((1, 1, 1, 3), lambda b, r: (b, r, 0, 0))
    topi_f, ps = pl.pallas_call(
        _rank_body,
        grid=grid,
        in_specs=[row_spec, col_spec, col_spec, col_spec],
        out_specs=(topi_spec, ps_spec),
        out_shape=(
            jax.ShapeDtypeStruct((B, 1, S), jnp.float32),
            jax.ShapeDtypeStruct((B, nb, 1, 3), jnp.float32),
        ),
    )(smax[:, None, :], smax[:, :, None],
      dist_loss[:, :, None], fmask[:, :, None])
    return topi_f.reshape(B, S), ps


def _gather1(arr, idx):
    return jax.vmap(lambda a, i: a[i])(arr, idx)


def kernel(x, pos, sample_W, sample_b):
    B, N = pos.shape[0], pos.shape[1]
    k = K
    fmask = _fps_pallas(jax.lax.stop_gradient(pos))
    fps_feature = (fmask - fmask.mean()) / fmask.sum()
    ip, d2 = _knn_feats(pos, x)
    ip = jnp.clip(ip, -1.0, 1.0)
    angle = jnp.arccos(ip)
    thr = math.pi / 2
    angle = jnp.where(angle > thr, math.pi - angle, angle)
    angle = angle.sum(axis=-1)
    curv = (angle - angle.mean()) / angle.sum()
    dists = jnp.sqrt(d2 + 1e-12)
    dmax = dists.max(axis=-1)
    dense = k / (dmax ** 3)
    inf_mask = jnp.isinf(dense)
    max_val = jnp.max(jnp.where(inf_mask, -jnp.inf, dense))
    dense = jnp.where(inf_mask, max_val, dense)
    dense = (dense - dense.mean()) / dense.sum()
    sampling_feats = jnp.stack([fps_feature, curv, dense], axis=-1)
    opt = (sampling_feats @ sample_W.T + sample_b)[..., 0]
    smax = jax.nn.softmax(opt, axis=1)
    dist_loss = dmax + dists.mean(axis=-1)
    topi_f, ps = _rank_pallas(smax, dist_loss, fmask)
    topi = topi_f.astype(jnp.int32)
    S = NUM_TO_SAMPLE
    shorl_mean = jnp.sum(ps[..., 0]) / (B * S)
    bdist_mean = jnp.sum(ps[..., 1]) / (B * S)
    total_loss = jnp.sum(ps[..., 2]) / (B * N)
    losses = jnp.stack([total_loss, total_loss, shorl_mean, bdist_mean])
    return topi, losses


# speedup vs baseline: 1.0662x; 1.0662x over previous
import math

import jax
import jax.numpy as jnp
from jax.experimental import pallas as pl

NUM_TO_SAMPLE = 1024
K = 10
_ROWS = 256

_ACOS_C = (
    1.5707963050,
    -0.2145988016,
    0.0889789874,
    -0.0501743046,
    0.0308918810,
    -0.0170881256,
    0.0066700901,
    -0.0012624911,
)


def _fps_body(px_ref, py_ref, pz_ref, out_ref, fmask_ref):
    B, N = px_ref.shape
    px = px_ref[...]
    py = py_ref[...]
    pz = pz_ref[...]
    iota = jax.lax.broadcasted_iota(jnp.int32, (B, N), 1)

    def step(s, state):
        dists, far, fmask = state
        out_ref[pl.ds(s, 1), :] = far[None, :]
        mask = iota == far[:, None]
        fmask = jnp.where(mask, 1.0, fmask)
        cx = jnp.sum(jnp.where(mask, px, 0.0), axis=1)
        cy = jnp.sum(jnp.where(mask, py, 0.0), axis=1)
        cz = jnp.sum(jnp.where(mask, pz, 0.0), axis=1)
        dx = px - cx[:, None]
        dy = py - cy[:, None]
        dz = pz - cz[:, None]
        d = dx * dx + dy * dy + dz * dz
        dists = jnp.minimum(dists, d)
        m = jnp.max(dists, axis=1)
        far_new = jnp.min(
            jnp.where(dists == m[:, None], iota, N), axis=1
        ).astype(jnp.int32)
        return dists, far_new, fmask

    dists0 = jnp.full((B, N), 1e10, dtype=jnp.float32)
    far0 = jnp.zeros((B,), dtype=jnp.int32)
    fmask0 = jnp.zeros((B, N), dtype=jnp.float32)
    _, _, fmask = jax.lax.fori_loop(
        0, NUM_TO_SAMPLE, step, (dists0, far0, fmask0)
    )
    fmask_ref[...] = fmask


def _fps_pallas(pos):
    B, N, _ = pos.shape
    p_idx_t, fmask = pl.pallas_call(
        _fps_body,
        out_shape=(
            jax.ShapeDtypeStruct((NUM_TO_SAMPLE, B), jnp.int32),
            jax.ShapeDtypeStruct((B, N), jnp.float32),
        ),
    )(pos[:, :, 0], pos[:, :, 1], pos[:, :, 2])
    return p_idx_t.T, fmask


def _feat_body(px_ref, py_ref, pz_ref, bx_ref, by_ref, bz_ref, xx_ref,
               pxi_ref, pyi_ref, pzi_ref, bxi_ref, byi_ref, bzi_ref,
               xxi_ref, xf0_ref, xf1_ref, xf2_ref,
               ip_ref, d2_ref):
    R = _ROWS
    N = px_ref.shape[2]
    pxj = px_ref[0, :, :]
    pyj = py_ref[0, :, :]
    pzj = pz_ref[0, :, :]
    bxj = bx_ref[0, :, :]
    byj = by_ref[0, :, :]
    bzj = bz_ref[0, :, :]
    xxj = xx_ref[0, :, :]
    pxi = pxi_ref[0, :, :]
    pyi = pyi_ref[0, :, :]
    pzi = pzi_ref[0, :, :]
    bxi = bxi_ref[0, :, :]
    byi = byi_ref[0, :, :]
    bzi = bzi_ref[0, :, :]
    xxi = xxi_ref[0, :, :]
    xf0 = xf0_ref[0, :, :]
    xf1 = xf1_ref[0, :, :]
    xf2 = xf2_ref[0, :, :]

    dx = pxi - pxj
    dy = pyi - pyj
    dz = pzi - pzj
    d2 = dx * dx + dy * dy + dz * dz
    g = xf0 * pxj + xf1 * pyj + xf2 * pzj
    m3 = (bxi * bxj + byi * byj) + bzi * bzj
    inner = -2.0 * m3
    key = ((-xxi) - inner) - xxj

    iota = jax.lax.broadcasted_iota(jnp.int32, (R, N), 1).astype(jnp.float32)
    big = jnp.float32(3.4e38)
    nf = jnp.float32(N)
    keyw = key
    for t in range(K):
        m = jnp.max(keyw, axis=1, keepdims=True)
        amin = jnp.min(
            jnp.where(keyw == m, iota, nf), axis=1, keepdims=True
        )
        sel = iota == amin
        ip_ref[0, 0, :, t] = jnp.sum(jnp.where(sel, g, 0.0), axis=1)
        d2_ref[0, 0, :, t] = jnp.sum(jnp.where(sel, d2, 0.0), axis=1)
        keyw = jnp.where(sel, -big, keyw)


def _knn_feats(pos, x):
    B, N, _ = pos.shape
    nb = N // _ROWS
    grid = (B, nb)
    posb = pos.astype(jnp.bfloat16).astype(jnp.float32)
    xx = jnp.sum(jnp.swapaxes(pos, 1, 2) ** 2, axis=1)
    row_spec = pl.BlockSpec((1, 1, N), lambda b, r: (b, 0, 0))
    col_spec = pl.BlockSpec((1, _ROWS, 1), lambda b, r: (b, r, 0))
    out_spec = pl.BlockSpec((1, 1, _ROWS, K), lambda b, r: (b, r, 0, 0))
    oshape = jax.ShapeDtypeStruct((B, nb, _ROWS, K), jnp.float32)
    ip, d2 = pl.pallas_call(
        _feat_body,
        grid=grid,
        in_specs=[row_spec] * 7 + [col_spec] * 10,
        out_specs=(out_spec, out_spec),
        out_shape=(oshape, oshape),
    )(pos[:, None, :, 0], pos[:, None, :, 1], pos[:, None, :, 2],
      posb[:, None, :, 0], posb[:, None, :, 1], posb[:, None, :, 2],
      xx[:, None, :],
      pos[:, :, 0:1], pos[:, :, 1:2], pos[:, :, 2:3],
      posb[:, :, 0:1], posb[:, :, 1:2], posb[:, :, 2:3],
      xx[:, :, None],
      x[:, 0, :, None], x[:, 1, :, None], x[:, 2, :, None])
    return ip.reshape(B, N, K), d2.reshape(B, N, K)


def _gather1(arr, idx):
    return jax.vmap(lambda a, i: a[i])(arr, idx)


def kernel(x, pos, sample_W, sample_b):
    B, N = pos.shape[0], pos.shape[1]
    k = K
    p_idx, fmask = _fps_pallas(jax.lax.stop_gradient(pos))
    fps_feature = (fmask - fmask.mean()) / fmask.sum()
    ip, d2 = _knn_feats(pos, x)
    ip = jnp.clip(ip, -1.0, 1.0)
    angle = jnp.arccos(ip)
    thr = math.pi / 2
    angle = jnp.where(angle > thr, math.pi - angle, angle)
    angle = angle.sum(axis=-1)
    curv = (angle - angle.mean()) / angle.sum()
    dists = jnp.sqrt(d2 + 1e-12)
    dmax = dists.max(axis=-1)
    dense = k / (dmax ** 3)
    inf_mask = jnp.isinf(dense)
    max_val = jnp.max(jnp.where(inf_mask, -jnp.inf, dense))
    dense = jnp.where(inf_mask, max_val, dense)
    dense = (dense - dense.mean()) / dense.sum()
    sampling_feats = jnp.stack([fps_feature, curv, dense], axis=-1)
    opt = (sampling_feats @ sample_W.T + sample_b)[..., 0]
    smax = jax.nn.softmax(opt, axis=1)
    topv, topi = jax.lax.top_k(smax, NUM_TO_SAMPLE)
    dist_loss = dmax + dists.mean(axis=-1)
    shorlisted_loss = _gather1(dist_loss, topi)
    sampling_loss = dist_loss * smax
    total_loss = sampling_loss.mean()
    bdist_loss = _gather1(dist_loss, p_idx)
    losses = jnp.stack(
        [total_loss, sampling_loss.mean(), shorlisted_loss.mean(), bdist_loss.mean()]
    )
    return topi, losses


# speedup vs baseline: 1.0708x; 1.0043x over previous
import math

import jax
import jax.numpy as jnp
from jax.experimental import pallas as pl

NUM_TO_SAMPLE = 1024
K = 10
_ROWS = 256

_ACOS_C = (
    1.5707963050,
    -0.2145988016,
    0.0889789874,
    -0.0501743046,
    0.0308918810,
    -0.0170881256,
    0.0066700901,
    -0.0012624911,
)


def _fps_body(px_ref, py_ref, pz_ref, fmask_ref):
    B, N = px_ref.shape
    px = px_ref[...]
    py = py_ref[...]
    pz = pz_ref[...]
    iota = jax.lax.broadcasted_iota(jnp.int32, (B, N), 1)

    def step(s, state):
        dists, far, fmask = state
        mask = iota == far[:, None]
        fmask = jnp.where(mask, 1.0, fmask)
        cx = jnp.sum(jnp.where(mask, px, 0.0), axis=1)
        cy = jnp.sum(jnp.where(mask, py, 0.0), axis=1)
        cz = jnp.sum(jnp.where(mask, pz, 0.0), axis=1)
        dx = px - cx[:, None]
        dy = py - cy[:, None]
        dz = pz - cz[:, None]
        d = dx * dx + dy * dy + dz * dz
        dists = jnp.minimum(dists, d)
        m = jnp.max(dists, axis=1)
        far_new = jnp.min(
            jnp.where(dists == m[:, None], iota, N), axis=1
        ).astype(jnp.int32)
        return dists, far_new, fmask

    dists0 = jnp.full((B, N), 1e10, dtype=jnp.float32)
    far0 = jnp.zeros((B,), dtype=jnp.int32)
    fmask0 = jnp.zeros((B, N), dtype=jnp.float32)
    _, _, fmask = jax.lax.fori_loop(
        0, NUM_TO_SAMPLE, step, (dists0, far0, fmask0)
    )
    fmask_ref[...] = fmask


def _fps_pallas(pos):
    B, N, _ = pos.shape
    fmask = pl.pallas_call(
        _fps_body,
        out_shape=jax.ShapeDtypeStruct((B, N), jnp.float32),
    )(pos[:, :, 0], pos[:, :, 1], pos[:, :, 2])
    return fmask


def _feat_body(px_ref, py_ref, pz_ref, bx_ref, by_ref, bz_ref, xx_ref,
               pxi_ref, pyi_ref, pzi_ref, bxi_ref, byi_ref, bzi_ref,
               xxi_ref, xf0_ref, xf1_ref, xf2_ref,
               ip_ref, d2_ref):
    R = _ROWS
    N = px_ref.shape[2]
    pxj = px_ref[0, :, :]
    pyj = py_ref[0, :, :]
    pzj = pz_ref[0, :, :]
    bxj = bx_ref[0, :, :]
    byj = by_ref[0, :, :]
    bzj = bz_ref[0, :, :]
    xxj = xx_ref[0, :, :]
    pxi = pxi_ref[0, :, :]
    pyi = pyi_ref[0, :, :]
    pzi = pzi_ref[0, :, :]
    bxi = bxi_ref[0, :, :]
    byi = byi_ref[0, :, :]
    bzi = bzi_ref[0, :, :]
    xxi = xxi_ref[0, :, :]
    xf0 = xf0_ref[0, :, :]
    xf1 = xf1_ref[0, :, :]
    xf2 = xf2_ref[0, :, :]

    dx = pxi - pxj
    dy = pyi - pyj
    dz = pzi - pzj
    d2 = dx * dx + dy * dy + dz * dz
    g = xf0 * pxj + xf1 * pyj + xf2 * pzj
    m3 = (bxi * bxj + byi * byj) + bzi * bzj
    inner = -2.0 * m3
    key = ((-xxi) - inner) - xxj

    iota = jax.lax.broadcasted_iota(jnp.int32, (R, N), 1).astype(jnp.float32)
    big = jnp.float32(3.4e38)
    nf = jnp.float32(N)
    keyw = key
    for t in range(K):
        m = jnp.max(keyw, axis=1, keepdims=True)
        amin = jnp.min(
            jnp.where(keyw == m, iota, nf), axis=1, keepdims=True
        )
        sel = iota == amin
        ip_ref[0, 0, :, t] = jnp.sum(jnp.where(sel, g, 0.0), axis=1)
        d2_ref[0, 0, :, t] = jnp.sum(jnp.where(sel, d2, 0.0), axis=1)
        keyw = jnp.where(sel, -big, keyw)


def _knn_feats(pos, x):
    B, N, _ = pos.shape
    nb = N // _ROWS
    grid = (B, nb)
    posb = pos.astype(jnp.bfloat16).astype(jnp.float32)
    xx = jnp.sum(jnp.swapaxes(pos, 1, 2) ** 2, axis=1)
    row_spec = pl.BlockSpec((1, 1, N), lambda b, r: (b, 0, 0))
    col_spec = pl.BlockSpec((1, _ROWS, 1), lambda b, r: (b, r, 0))
    out_spec = pl.BlockSpec((1, 1, _ROWS, K), lambda b, r: (b, r, 0, 0))
    oshape = jax.ShapeDtypeStruct((B, nb, _ROWS, K), jnp.float32)
    ip, d2 = pl.pallas_call(
        _feat_body,
        grid=grid,
        in_specs=[row_spec] * 7 + [col_spec] * 10,
        out_specs=(out_spec, out_spec),
        out_shape=(oshape, oshape),
    )(pos[:, None, :, 0], pos[:, None, :, 1], pos[:, None, :, 2],
      posb[:, None, :, 0], posb[:, None, :, 1], posb[:, None, :, 2],
      xx[:, None, :],
      pos[:, :, 0:1], pos[:, :, 1:2], pos[:, :, 2:3],
      posb[:, :, 0:1], posb[:, :, 1:2], posb[:, :, 2:3],
      xx[:, :, None],
      x[:, 0, :, None], x[:, 1, :, None], x[:, 2, :, None])
    return ip.reshape(B, N, K), d2.reshape(B, N, K)


def _gather1(arr, idx):
    return jax.vmap(lambda a, i: a[i])(arr, idx)


def kernel(x, pos, sample_W, sample_b):
    B, N = pos.shape[0], pos.shape[1]
    k = K
    fmask = _fps_pallas(jax.lax.stop_gradient(pos))
    fps_feature = (fmask - fmask.mean()) / fmask.sum()
    ip, d2 = _knn_feats(pos, x)
    ip = jnp.clip(ip, -1.0, 1.0)
    angle = jnp.arccos(ip)
    thr = math.pi / 2
    angle = jnp.where(angle > thr, math.pi - angle, angle)
    angle = angle.sum(axis=-1)
    curv = (angle - angle.mean()) / angle.sum()
    dists = jnp.sqrt(d2 + 1e-12)
    dmax = dists.max(axis=-1)
    dense = k / (dmax ** 3)
    inf_mask = jnp.isinf(dense)
    max_val = jnp.max(jnp.where(inf_mask, -jnp.inf, dense))
    dense = jnp.where(inf_mask, max_val, dense)
    dense = (dense - dense.mean()) / dense.sum()
    sampling_feats = jnp.stack([fps_feature, curv, dense], axis=-1)
    opt = (sampling_feats @ sample_W.T + sample_b)[..., 0]
    smax = jax.nn.softmax(opt, axis=1)
    topv, topi = jax.lax.top_k(smax, NUM_TO_SAMPLE)
    dist_loss = dmax + dists.mean(axis=-1)
    shorlisted_loss = _gather1(dist_loss, topi)
    sampling_loss = dist_loss * smax
    total_loss = sampling_loss.mean()
    bdist_mean = jnp.sum(dist_loss * fmask) / (B * NUM_TO_SAMPLE)
    losses = jnp.stack(
        [total_loss, sampling_loss.mean(), shorlisted_loss.mean(), bdist_mean]
    )
    return topi, losses
